# Initial kernel scaffold; baseline (speedup 1.0000x reference)
import jax
import jax.numpy as jnp
from jax.experimental import pallas as pl


def kernel(h, graph_ids):
    raise NotImplementedError("write your pallas kernel here")


# speedup vs baseline: 4.0629x; 4.0629x over previous
import functools

import jax
import jax.numpy as jnp
from jax import lax
from jax.experimental import pallas as pl
from jax.experimental.pallas import tpu as pltpu
from jax.experimental.pallas import tpu_sc as plsc

NUM_SEGMENTS = 256
N_ROWS = 50000
D = 512
NC = 2
NS = 16
DC = D // NC
B = 125
NB = N_ROWS // B
BLOCKS_PER_TILE = NB // NS
SEGS_PER_TILE = NUM_SEGMENTS // NS
L = 16


def _seg_mean_body(h_hbm, ids_hbm, out_hbm,
                   hblk, ids_v, ones_v, zer_v, zcnt_v, sums_v, cnt_v, out_v,
                   sums_sh, cnt_sh):
    core = lax.axis_index("c")
    sid = lax.axis_index("s")
    col0 = core * DC

    zero16 = jnp.zeros((L,), jnp.float32)
    one16 = jnp.ones((L,), jnp.float32)
    for r in range(B):
        ones_v[r, :] = one16
    for r in range(SEGS_PER_TILE):
        zcnt_v[r, :] = zero16
        for j in range(DC // L):
            zer_v[r, pl.ds(j * L, L)] = zero16

    seg0 = sid * SEGS_PER_TILE
    pltpu.sync_copy(zer_v, sums_sh.at[pl.ds(seg0, SEGS_PER_TILE)])
    pltpu.sync_copy(zcnt_v, cnt_sh.at[pl.ds(seg0, SEGS_PER_TILE)])
    plsc.subcore_barrier()

    def block_step(i, carry):
        b = sid * BLOCKS_PER_TILE + i
        pltpu.sync_copy(ids_hbm.at[b], ids_v)
        pltpu.sync_copy(h_hbm.at[pl.ds(b * B, B), pl.ds(col0, DC)], hblk)
        pltpu.sync_copy(hblk, sums_sh.at[ids_v], add=True)
        pltpu.sync_copy(ones_v, cnt_sh.at[ids_v], add=True)
        return carry

    lax.fori_loop(0, BLOCKS_PER_TILE, block_step, 0)
    plsc.subcore_barrier()

    pltpu.sync_copy(sums_sh.at[pl.ds(seg0, SEGS_PER_TILE)], sums_v)
    pltpu.sync_copy(cnt_sh.at[pl.ds(seg0, SEGS_PER_TILE)], cnt_v)
    for r in range(SEGS_PER_TILE):
        recip = 1.0 / jnp.maximum(cnt_v[r, :], 1.0)
        for j in range(DC // L):
            out_v[r, pl.ds(j * L, L)] = sums_v[r, pl.ds(j * L, L)] * recip
    pltpu.sync_copy(out_v, out_hbm.at[pl.ds(seg0, SEGS_PER_TILE), pl.ds(col0, DC)])


@jax.jit
def _seg_mean(h, ids2d):
    mesh = plsc.VectorSubcoreMesh(
        core_axis_name="c", subcore_axis_name="s", num_cores=NC, num_subcores=NS
    )
    k = pl.kernel(
        _seg_mean_body,
        out_type=jax.ShapeDtypeStruct((NUM_SEGMENTS, D), jnp.float32),
        mesh=mesh,
        compiler_params=pltpu.CompilerParams(use_tc_tiling_on_sc=False),
        scratch_types=[
            pltpu.VMEM((B, DC), jnp.float32),
            pltpu.VMEM((B,), jnp.int32),
            pltpu.VMEM((B, L), jnp.float32),
            pltpu.VMEM((SEGS_PER_TILE, DC), jnp.float32),
            pltpu.VMEM((SEGS_PER_TILE, L), jnp.float32),
            pltpu.VMEM((SEGS_PER_TILE, DC), jnp.float32),
            pltpu.VMEM((SEGS_PER_TILE, L), jnp.float32),
            pltpu.VMEM((SEGS_PER_TILE, DC), jnp.float32),
            pltpu.VMEM_SHARED((NUM_SEGMENTS, DC), jnp.float32),
            pltpu.VMEM_SHARED((NUM_SEGMENTS, L), jnp.float32),
        ],
    )
    return k(h, ids2d)


def kernel(h, graph_ids):
    ids2d = graph_ids.astype(jnp.int32).reshape(NB, B)
    return _seg_mean(h, ids2d)


# speedup vs baseline: 4.7643x; 1.1726x over previous
import functools

import jax
import jax.numpy as jnp
from jax import lax
from jax.experimental import pallas as pl
from jax.experimental.pallas import tpu as pltpu
from jax.experimental.pallas import tpu_sc as plsc

NUM_SEGMENTS = 256
N_ROWS = 50000
D = 512
NC = 2
NS = 16
DC = D // NC
B = 125
NB = N_ROWS // B
BLOCKS_PER_TILE = NB // NS
SEGS_PER_TILE = NUM_SEGMENTS // NS
L = 16


def _seg_mean_body(h_hbm, ids_hbm, out_hbm,
                   hblk0, hblk1, ids_v, ones_v, zer_v, zcnt_v,
                   sums_v, cnt_v, out_v, sem0, sem1,
                   sums_sh, cnt_sh):
    core = lax.axis_index("c")
    sid = lax.axis_index("s")
    col0 = core * DC
    hbufs = (hblk0, hblk1)
    sems = (sem0, sem1)

    zero16 = jnp.zeros((L,), jnp.float32)
    one16 = jnp.ones((L,), jnp.float32)
    for r in range(B):
        ones_v[r, :] = one16
    for r in range(SEGS_PER_TILE):
        zcnt_v[r, :] = zero16
        for j in range(DC // L):
            zer_v[r, pl.ds(j * L, L)] = zero16

    seg0 = sid * SEGS_PER_TILE
    b0 = sid * BLOCKS_PER_TILE
    pltpu.sync_copy(zer_v, sums_sh.at[pl.ds(seg0, SEGS_PER_TILE)])
    pltpu.sync_copy(zcnt_v, cnt_sh.at[pl.ds(seg0, SEGS_PER_TILE)])
    pltpu.sync_copy(ids_hbm.at[pl.ds(b0, BLOCKS_PER_TILE)], ids_v)
    plsc.subcore_barrier()

    def gather(i, buf, sem):
        rows = pl.ds((b0 + i) * B, B)
        return pltpu.async_copy(h_hbm.at[rows, pl.ds(col0, DC)], buf, sem)

    pending = gather(0, hbufs[0], sems[0])
    for i in range(BLOCKS_PER_TILE):
        cur = i % 2
        pending.wait()
        if i + 1 < BLOCKS_PER_TILE:
            pending = gather(i + 1, hbufs[1 - cur], sems[1 - cur])
        pltpu.sync_copy(hbufs[cur], sums_sh.at[ids_v.at[i]], add=True)
        pltpu.sync_copy(ones_v, cnt_sh.at[ids_v.at[i]], add=True)
    plsc.subcore_barrier()

    pltpu.sync_copy(sums_sh.at[pl.ds(seg0, SEGS_PER_TILE)], sums_v)
    pltpu.sync_copy(cnt_sh.at[pl.ds(seg0, SEGS_PER_TILE)], cnt_v)
    for r in range(SEGS_PER_TILE):
        recip = 1.0 / jnp.maximum(cnt_v[r, :], 1.0)
        for j in range(DC // L):
            out_v[r, pl.ds(j * L, L)] = sums_v[r, pl.ds(j * L, L)] * recip
    pltpu.sync_copy(out_v, out_hbm.at[pl.ds(seg0, SEGS_PER_TILE), pl.ds(col0, DC)])


@jax.jit
def _seg_mean(h, ids2d):
    mesh = plsc.VectorSubcoreMesh(
        core_axis_name="c", subcore_axis_name="s", num_cores=NC, num_subcores=NS
    )
    k = pl.kernel(
        _seg_mean_body,
        out_type=jax.ShapeDtypeStruct((NUM_SEGMENTS, D), jnp.float32),
        mesh=mesh,
        compiler_params=pltpu.CompilerParams(use_tc_tiling_on_sc=False),
        scratch_types=[
            pltpu.VMEM((B, DC), jnp.float32),
            pltpu.VMEM((B, DC), jnp.float32),
            pltpu.VMEM((BLOCKS_PER_TILE, B), jnp.int32),
            pltpu.VMEM((B, L), jnp.float32),
            pltpu.VMEM((SEGS_PER_TILE, DC), jnp.float32),
            pltpu.VMEM((SEGS_PER_TILE, L), jnp.float32),
            pltpu.VMEM((SEGS_PER_TILE, DC), jnp.float32),
            pltpu.VMEM((SEGS_PER_TILE, L), jnp.float32),
            pltpu.VMEM((SEGS_PER_TILE, DC), jnp.float32),
            pltpu.SemaphoreType.DMA,
            pltpu.SemaphoreType.DMA,
            pltpu.VMEM_SHARED((NUM_SEGMENTS, DC), jnp.float32),
            pltpu.VMEM_SHARED((NUM_SEGMENTS, L), jnp.float32),
        ],
    )
    return k(h, ids2d)


def kernel(h, graph_ids):
    ids2d = graph_ids.astype(jnp.int32).reshape(NB, B)
    return _seg_mean(h, ids2d)
